# baseline (speedup 1.0000x reference)
import jax
import jax.numpy as jnp
from jax import lax
from jax.experimental import pallas as pl
from jax.experimental.pallas import tpu as pltpu
from jax.experimental.pallas import tpu_sc as plsc

N = 10000
E = 320000
EH = E // 2
D_FEAT = 128
D_EDGE = 16
HID = 128
B = 128
FFN_HID = 256

_NC, _NS = 2, 16
_NW = _NC * _NS
_CH = 80
_EPT = E // _NW
_CPT = _EPT // _CH

_RE = 2000
_NBE = E // _RE
_NBH = EH // _RE
_RN = 2000
_NBN = N // _RN

_mesh = plsc.VectorSubcoreMesh(
    core_axis_name="c", subcore_axis_name="s", num_cores=_NC, num_subcores=_NS)

_f32 = jnp.float32


_GK = 5
_NG = _CPT // _GK
_GR = _GK * _CH


def _sc_gather_body(table, idx, out, idxb, buf_a, buf_b, semg, semw_a, semw_b):
    w = lax.axis_index("c") * _NS + lax.axis_index("s")
    pltpu.sync_copy(idx.at[w], idxb)
    out0 = w * _EPT

    def group(i, buf, semw, first):
        @pl.when(jnp.logical_not(first))
        def _():
            pltpu.make_async_copy(buf, out.at[pl.ds(out0, _GR)], semw).wait()
        descs = [
            pltpu.async_copy(table.at[idxb.at[i * _GK + b]],
                             buf.at[pl.ds(b * _CH, _CH)], semg)
            for b in range(_GK)
        ]
        for dsc in descs:
            dsc.wait()
        pltpu.async_copy(buf, out.at[pl.ds(out0 + i * _GR, _GR)], semw)

    def body(i, carry):
        even = (i % 2) == 0

        @pl.when(even)
        def _():
            group(i, buf_a, semw_a, i == 0)

        @pl.when(jnp.logical_not(even))
        def _():
            group(i, buf_b, semw_b, i == 1)

        return carry

    lax.fori_loop(0, _NG, body, 0)
    pltpu.make_async_copy(buf_a, out.at[pl.ds(out0, _GR)], semw_a).wait()
    pltpu.make_async_copy(buf_b, out.at[pl.ds(out0, _GR)], semw_b).wait()


def _sc_gather(table, idx3):
    fn = pl.kernel(
        _sc_gather_body,
        out_type=jax.ShapeDtypeStruct((E, HID), _f32),
        mesh=_mesh,
        scratch_types=[
            pltpu.VMEM((_CPT, _CH), jnp.int32),
            pltpu.VMEM((_GR, HID), _f32),
            pltpu.VMEM((_GR, HID), _f32),
            pltpu.SemaphoreType.DMA,
            pltpu.SemaphoreType.DMA,
            pltpu.SemaphoreType.DMA,
        ],
    )
    return fn(table, idx3)


def _sc_segsum_body(rows, idx, zeros, out, idxb, buf_a, buf_b, acc,
                    semr_a, semr_b, semsc):
    c = lax.axis_index("c")
    s = lax.axis_index("s")
    w = c * _NS + s

    @pl.when(s < _NS - 1)
    def _():
        pltpu.sync_copy(zeros.at[pl.ds(s * 624, 624)], acc.at[pl.ds(s * 624, 624)])

    @pl.when(s == _NS - 1)
    def _():
        pltpu.sync_copy(zeros.at[pl.ds(624 * 15, 640)], acc.at[pl.ds(624 * 15, 640)])

    pltpu.sync_copy(idx.at[w], idxb)
    plsc.subcore_barrier()

    in0 = w * _EPT
    pltpu.async_copy(rows.at[pl.ds(in0, _CH)], buf_a, semr_a)

    def chunk(i, buf, semr, nbuf, nsemr):
        pltpu.make_async_copy(rows.at[pl.ds(in0, _CH)], buf, semr).wait()

        @pl.when(i > 0)
        def _():
            pltpu.make_async_copy(nbuf, acc.at[idxb.at[i]], semsc).wait()

        @pl.when(i + 1 < _CPT)
        def _():
            pltpu.async_copy(rows.at[pl.ds(in0 + (i + 1) * _CH, _CH)], nbuf, nsemr)

        pltpu.async_copy(buf, acc.at[idxb.at[i]], semsc, add=True)

    def body(i, carry):
        even = (i % 2) == 0

        @pl.when(even)
        def _():
            chunk(i, buf_a, semr_a, buf_b, semr_b)

        @pl.when(jnp.logical_not(even))
        def _():
            chunk(i, buf_b, semr_b, buf_a, semr_a)

        return carry

    lax.fori_loop(0, _CPT, body, 0)
    pltpu.make_async_copy(buf_a, acc.at[idxb.at[_CPT - 1]], semsc).wait()
    plsc.subcore_barrier()

    @pl.when(s < _NS - 1)
    def _():
        pltpu.sync_copy(acc.at[pl.ds(s * 624, 624)],
                        out.at[c, pl.ds(s * 624, 624)])

    @pl.when(s == _NS - 1)
    def _():
        pltpu.sync_copy(acc.at[pl.ds(624 * 15, 640)],
                        out.at[c, pl.ds(624 * 15, 640)])


def _sc_segsum(rows, idx3, zeros):
    fn = pl.kernel(
        _sc_segsum_body,
        out_type=jax.ShapeDtypeStruct((_NC, N, HID), _f32),
        mesh=_mesh,
        scratch_types=[
            pltpu.VMEM((_CPT, _CH), jnp.int32),
            pltpu.VMEM((_CH, HID), _f32),
            pltpu.VMEM((_CH, HID), _f32),
            pltpu.VMEM_SHARED((N, HID), _f32),
            pltpu.SemaphoreType.DMA,
            pltpu.SemaphoreType.DMA,
            pltpu.SemaphoreType.DMA,
        ],
    )
    return fn(rows, idx3, zeros)


def _merge_body(p_ref, out_ref):
    out_ref[...] = p_ref[0] + p_ref[1]


def _tc_merge(p):
    return pl.pallas_call(
        _merge_body,
        grid=(_NBN,),
        in_specs=[pl.BlockSpec((_NC, _RN, HID), lambda i: (0, i, 0))],
        out_specs=pl.BlockSpec((_RN, HID), lambda i: (i, 0)),
        out_shape=jax.ShapeDtypeStruct((N, HID), _f32),
    )(p)


def _edge0_body(ea_ref, xs_ref, wi_ref, h0_ref):
    cat = jnp.concatenate([xs_ref[...], ea_ref[...]], axis=1)
    h0_ref[...] = jnp.maximum(
        jnp.dot(cat, wi_ref[...], preferred_element_type=_f32), 0.0)


def _tc_edge0(ea_di, xsrc, w_i):
    return pl.pallas_call(
        _edge0_body,
        grid=(_NBE,),
        in_specs=[
            pl.BlockSpec((_RE, D_EDGE), lambda i: (i, 0)),
            pl.BlockSpec((_RE, HID), lambda i: (i, 0)),
            pl.BlockSpec((D_FEAT + D_EDGE, HID), lambda i: (0, 0)),
        ],
        out_specs=pl.BlockSpec((_RE, HID), lambda i: (i, 0)),
        out_shape=jax.ShapeDtypeStruct((E, HID), _f32),
    )(ea_di, xsrc, w_i)


def _step_body(h0_ref, gs_ref, hsw_ref, wh_ref, out_ref):
    m = gs_ref[...] - hsw_ref[...]
    out_ref[...] = jnp.maximum(
        h0_ref[...] + jnp.dot(m, wh_ref[...], preferred_element_type=_f32), 0.0)


def _tc_step(h0, gs, h, w_h):
    return pl.pallas_call(
        _step_body,
        grid=(_NBE,),
        in_specs=[
            pl.BlockSpec((_RE, HID), lambda i: (i, 0)),
            pl.BlockSpec((_RE, HID), lambda i: (i, 0)),
            pl.BlockSpec((_RE, HID), lambda i: ((i + _NBH) % _NBE, 0)),
            pl.BlockSpec((HID, HID), lambda i: (0, 0)),
        ],
        out_specs=pl.BlockSpec((_RE, HID), lambda i: (i, 0)),
        out_shape=jax.ShapeDtypeStruct((E, HID), _f32),
    )(h0, gs, h, w_h)


def _nodeout_body(x_ref, nm_ref, wo_ref, out_ref):
    nm = nm_ref[0] + nm_ref[1]
    cat = jnp.concatenate([x_ref[...], nm], axis=1)
    out_ref[...] = jnp.maximum(
        jnp.dot(cat, wo_ref[...], preferred_element_type=_f32), 0.0)


def _tc_nodeout(x, nm, w_o):
    return pl.pallas_call(
        _nodeout_body,
        grid=(_NBN,),
        in_specs=[
            pl.BlockSpec((_RN, D_FEAT), lambda i: (i, 0)),
            pl.BlockSpec((_NC, _RN, HID), lambda i: (0, i, 0)),
            pl.BlockSpec((D_FEAT + HID, HID), lambda i: (0, 0)),
        ],
        out_specs=pl.BlockSpec((_RN, HID), lambda i: (i, 0)),
        out_shape=jax.ShapeDtypeStruct((N, HID), _f32),
    )(x, nm, w_o)


def _pool_body(hv_ref, b_ref, sums_ref, cnt_ref):
    i = pl.program_id(0)

    @pl.when(i == 0)
    def _():
        sums_ref[...] = jnp.zeros_like(sums_ref)
        cnt_ref[...] = jnp.zeros_like(cnt_ref)

    ids = b_ref[0, 0, :]
    oneh = (ids[:, None] == lax.broadcasted_iota(jnp.int32, (_RN, B), 1)).astype(_f32)
    dn = (((0,), (0,)), ((), ()))
    sums_ref[...] += lax.dot_general(oneh, hv_ref[...], dn,
                                     preferred_element_type=_f32,
                                     precision=jax.lax.Precision.HIGHEST)
    cnt_ref[...] += lax.dot_general(oneh, jnp.ones((_RN, HID), _f32), dn,
                                    preferred_element_type=_f32,
                                    precision=jax.lax.Precision.HIGHEST)


def _tc_pool(hv, batch3):
    return pl.pallas_call(
        _pool_body,
        grid=(_NBN,),
        in_specs=[
            pl.BlockSpec((_RN, HID), lambda i: (i, 0)),
            pl.BlockSpec((1, 1, _RN), lambda i: (i, 0, 0)),
        ],
        out_specs=[
            pl.BlockSpec((B, HID), lambda i: (0, 0)),
            pl.BlockSpec((B, HID), lambda i: (0, 0)),
        ],
        out_shape=[
            jax.ShapeDtypeStruct((B, HID), _f32),
            jax.ShapeDtypeStruct((B, HID), _f32),
        ],
    )(hv, batch3)


def _ffn_body(s1, c1, s2, c2, w1, b1, w2, b2, w3, b3, out):
    v1 = s1[...] / jnp.maximum(c1[...], 1.0)
    v2 = s2[...] / jnp.maximum(c2[...], 1.0)
    v = jnp.concatenate([v1, v2], axis=1)
    h = jnp.maximum(jnp.dot(v, w1[...], preferred_element_type=_f32) + b1[...], 0.0)
    h = jnp.maximum(jnp.dot(h, w2[...], preferred_element_type=_f32) + b2[...], 0.0)
    out[...] = jnp.dot(h, w3[...], preferred_element_type=_f32) + b3[...]


def _tc_ffn(s1, c1, s2, c2, fw1, fb1, fw2, fb2, fw3, fb3):
    def full(shape):
        return pl.BlockSpec(shape, lambda: tuple(0 for _ in shape))
    return pl.pallas_call(
        _ffn_body,
        in_specs=[
            full((B, HID)), full((B, HID)), full((B, HID)), full((B, HID)),
            full((2 * HID, FFN_HID)), full((1, FFN_HID)),
            full((FFN_HID, FFN_HID)), full((1, FFN_HID)),
            full((FFN_HID, 1)), full((1, 1)),
        ],
        out_specs=full((B, 1)),
        out_shape=jax.ShapeDtypeStruct((B, 1), _f32),
    )(s1, c1, s2, c2, fw1, fb1.reshape(1, -1), fw2, fb2.reshape(1, -1),
      fw3, fb3.reshape(1, 1))


def _mpn(x, ei, ea, batch, w_i, w_h, w_o, zeros):
    s = ei[0, 0::2]
    d = ei[1, 0::2]
    idxg3 = jnp.concatenate([s, d]).reshape(_NW, _CPT, _CH)
    idxs3 = jnp.concatenate([d, s]).reshape(_NW, _CPT, _CH)
    ea_di = jnp.concatenate([ea[0::2], ea[1::2]])

    xsrc = _sc_gather(x, idxg3)
    h0 = _tc_edge0(ea_di, xsrc, w_i)

    h = h0
    for _ in range(2):
        p = _sc_segsum(h, idxs3, zeros)
        gs = _sc_gather(_tc_merge(p), idxg3)
        h = _tc_step(h0, gs, h, w_h)

    p = _sc_segsum(h, idxs3, zeros)
    hv = _tc_nodeout(x, p, w_o)
    return _tc_pool(hv, batch.reshape(_NBN, 1, _RN))


def kernel(x1, edge_index1, edge_attr1, batch1,
           x2, edge_index2, edge_attr2, batch2,
           W_i1, W_h1, W_o1, W_i2, W_h2, W_o2,
           ffn_W1, ffn_b1, ffn_W2, ffn_b2, ffn_W3, ffn_b3):
    zeros = jnp.zeros((N, HID), _f32)
    s1, c1 = _mpn(x1, edge_index1, edge_attr1, batch1, W_i1, W_h1, W_o1, zeros)
    s2, c2 = _mpn(x2, edge_index2, edge_attr2, batch2, W_i2, W_h2, W_o2, zeros)
    return _tc_ffn(s1, c1, s2, c2, ffn_W1, ffn_b1, ffn_W2, ffn_b2, ffn_W3, ffn_b3)
